# Initial kernel scaffold; baseline (speedup 1.0000x reference)
import jax
import jax.numpy as jnp
from jax.experimental import pallas as pl


def kernel(x, xyz, params, edge_index):
    raise NotImplementedError("write your pallas kernel here")


# speedup vs baseline: 3.2935x; 3.2935x over previous
import functools

import jax
import jax.numpy as jnp
from jax import lax
from jax.experimental import pallas as pl
from jax.experimental.pallas import tpu as pltpu
from jax.experimental.pallas import tpu_sc as plsc

N = 16384
E = 262144
D = 256
B = 32
EPS = 1e-5

NC = 2
NS = 16
NW = NC * NS

NPT = 256
NPASS = N // (NW * NPT)
CHUNK = 2048
NCH = E // CHUNK
CAP = 12288
BLK = 2048
SUB = 64
PADS = 2 * SUB

RT = 512
PREC = lax.Precision.HIGHEST

_LANE = None


def _prefix_sum16(v):
    lane = lax.iota(jnp.int32, 16)
    for k in (1, 2, 4, 8):
        idx = jnp.maximum(lane - k, 0)
        shifted = jnp.where(lane >= k, v[idx], 0)
        v = v + shifted
    return v


def _partition_body(srcdst_hbm, ls_hbm, ld_hbm, cnt_hbm,
                    edge0, edge1, lsa, lda, lsb, ldb, cbuf, sem0, sem1):
    wid = lax.axis_index("s") * NC + lax.axis_index("c")
    owner_a = wid
    owner_b = NW + wid

    def start_load(ch, par):
        sem = (sem0, sem1)[par]
        buf = (edge0, edge1)[par]
        return pltpu.async_copy(srcdst_hbm.at[pl.ds(2 * ch, 2)], buf, sem)

    def process(par, carry):
        buf = (edge0, edge1)[par]

        def scan_step(v, carry):
            cnt_a, cnt_b = carry
            d16 = buf[1, pl.ds(v * 16, 16)]
            s16 = buf[0, pl.ds(v * 16, 16)]
            g = d16 >> 8
            dl = d16 & (NPT - 1)
            ma = g == owner_a
            mb = g == owner_b
            psa = _prefix_sum16(ma.astype(jnp.int32))
            psb = _prefix_sum16(mb.astype(jnp.int32))
            pos_a = jnp.minimum(psa + (cnt_a - 1), CAP - 1)
            pos_b = jnp.minimum(psb + (cnt_b - 1), CAP - 1)
            plsc.store_scatter(lsa, [pos_a], s16, mask=ma)
            plsc.store_scatter(lda, [pos_a], dl, mask=ma)
            plsc.store_scatter(lsb, [pos_b], s16, mask=mb)
            plsc.store_scatter(ldb, [pos_b], dl, mask=mb)
            return (jnp.minimum(cnt_a + psa[15], CAP - PADS),
                    jnp.minimum(cnt_b + psb[15], CAP - PADS))

        return lax.fori_loop(0, CHUNK // 16, scan_step, carry)

    cp0 = start_load(0, 0)
    cp1 = start_load(1, 1)

    def pair_step(pc, carry):
        ch0 = 2 * pc
        cp0.wait()
        carry = process(0, carry)
        start_load(jnp.minimum(ch0 + 2, NCH - 2), 0)
        cp1.wait()
        carry = process(1, carry)
        start_load(jnp.minimum(ch0 + 3, NCH - 1), 1)
        return carry

    cnt_a, cnt_b = lax.fori_loop(0, NCH // 2, pair_step, (0, 0))
    cp0.wait()
    cp1.wait()

    pltpu.sync_copy(lsa, ls_hbm.at[pl.ds((0 * NW + wid) * CAP, CAP)])
    pltpu.sync_copy(lda, ld_hbm.at[pl.ds((0 * NW + wid) * CAP, CAP)])
    pltpu.sync_copy(lsb, ls_hbm.at[pl.ds((1 * NW + wid) * CAP, CAP)])
    pltpu.sync_copy(ldb, ld_hbm.at[pl.ds((1 * NW + wid) * CAP, CAP)])

    cbuf[pl.ds(0, 16)] = jnp.zeros((16,), jnp.int32) + cnt_a
    cbuf[pl.ds(16, 16)] = jnp.zeros((16,), jnp.int32) + cnt_b
    pltpu.sync_copy(cbuf.at[pl.ds(0, 16)],
                    cnt_hbm.at[pl.ds((0 * NW + wid) * 16, 16)])
    pltpu.sync_copy(cbuf.at[pl.ds(16, 16)],
                    cnt_hbm.at[pl.ds((1 * NW + wid) * 16, 16)])


@functools.cache
def _partition_kernel():
    return pl.kernel(
        _partition_body,
        out_type=(
            jax.ShapeDtypeStruct((NPASS * NW * CAP,), jnp.int32),
            jax.ShapeDtypeStruct((NPASS * NW * CAP,), jnp.int32),
            jax.ShapeDtypeStruct((NPASS * NW * 16,), jnp.int32),
        ),
        mesh=plsc.VectorSubcoreMesh(core_axis_name="c", subcore_axis_name="s",
                                    num_cores=NC, num_subcores=NS),
        compiler_params=pltpu.CompilerParams(needs_layout_passes=False),
        scratch_types=[
            pltpu.VMEM((2, CHUNK), jnp.int32),
            pltpu.VMEM((2, CHUNK), jnp.int32),
            pltpu.VMEM((CAP,), jnp.int32),
            pltpu.VMEM((CAP,), jnp.int32),
            pltpu.VMEM((CAP,), jnp.int32),
            pltpu.VMEM((CAP,), jnp.int32),
            pltpu.VMEM((32,), jnp.int32),
            pltpu.SemaphoreType.DMA,
            pltpu.SemaphoreType.DMA,
        ],
    )


def _segmax_body(t_hbm, ls_hbm, ld_hbm, cnt_hbm, out_hbm,
                 acc, bs, bd, cbuf, rows0, rows1, semb, semg0, semg1):
    wid = lax.axis_index("s") * NC + lax.axis_index("c")
    neg = jnp.full((16,), -jnp.inf, dtype=jnp.float32)

    for p in range(NPASS):
        owner = p * NW + wid
        base = owner * NPT
        lbase = (p * NW + wid) * CAP

        def init_row(r, _):
            for c in range(D // 16):
                acc[r, pl.ds(c * 16, 16)] = neg
            return 0
        lax.fori_loop(0, NPT, init_row, 0)

        pltpu.sync_copy(cnt_hbm.at[pl.ds((p * NW + wid) * 16, 16)],
                        cbuf.at[pl.ds(0, 16)])
        cnt = cbuf[pl.ds(0, 16)][0]
        nblk = (cnt + BLK - 1) // BLK

        def rmw_batch(rows, s0, lcnt):
            mb = jnp.clip(lcnt - s0, 0, SUB)

            def rmw(e, _):
                dl = bd[pl.ds(s0 + e, 16)][0]
                for c in range(D // 16):
                    sl = pl.ds(c * 16, 16)
                    acc[dl, sl] = jnp.maximum(acc[dl, sl], rows[e, sl])
                return 0
            lax.fori_loop(0, mb, rmw, 0)

        def blk_step(bq, _):
            b0 = bq * BLK
            pltpu.sync_copy(ls_hbm.at[pl.ds(lbase + b0, BLK)],
                            bs.at[pl.ds(0, BLK)])
            pltpu.sync_copy(ld_hbm.at[pl.ds(lbase + b0, BLK)],
                            bd.at[pl.ds(0, BLK)])
            lcnt = jnp.minimum(cnt - b0, BLK)

            pad = jnp.full((16,), base, dtype=jnp.int32)
            for j in range(PADS // 16):
                bs[pl.ds(lcnt + j * 16, 16)] = pad

            nsub = (lcnt + SUB - 1) // SUB
            npair = (nsub + 1) // 2

            def gather(sb, rows, sem):
                s0 = jnp.minimum(sb, 2 * npair - 1) * SUB
                return pltpu.async_copy(t_hbm.at[bs.at[pl.ds(s0, SUB)]],
                                        rows, sem)

            cp0 = gather(0, rows0, semg0)
            cp1 = gather(1, rows1, semg1)

            def pair_step(pc, _):
                s0 = 2 * pc * SUB
                cp0.wait()
                rmw_batch(rows0, s0, lcnt)
                gather(2 * pc + 2, rows0, semg0)
                cp1.wait()
                rmw_batch(rows1, s0 + SUB, lcnt)
                gather(2 * pc + 3, rows1, semg1)
                return 0
            lax.fori_loop(0, npair, pair_step, 0)
            cp0.wait()
            cp1.wait()
            return 0
        lax.fori_loop(0, nblk, blk_step, 0)

        pltpu.sync_copy(acc, out_hbm.at[pl.ds(base, NPT)])


@functools.cache
def _segmax_kernel():
    return pl.kernel(
        _segmax_body,
        out_type=jax.ShapeDtypeStruct((N, D), jnp.float32),
        mesh=plsc.VectorSubcoreMesh(core_axis_name="c", subcore_axis_name="s",
                                    num_cores=NC, num_subcores=NS),
        compiler_params=pltpu.CompilerParams(needs_layout_passes=False),
        scratch_types=[
            pltpu.VMEM((NPT, D), jnp.float32),
            pltpu.VMEM((BLK + PADS + 16,), jnp.int32),
            pltpu.VMEM((BLK + 16,), jnp.int32),
            pltpu.VMEM((16,), jnp.int32),
            pltpu.VMEM((SUB, D), jnp.float32),
            pltpu.VMEM((SUB, D), jnp.float32),
            pltpu.SemaphoreType.DMA,
            pltpu.SemaphoreType.DMA,
            pltpu.SemaphoreType.DMA,
        ],
    )


def _pre_body(scale, shift, h, tw, pw, bias, t_out, u_out):
    hn = h[...] * scale[...] + shift[...]
    t = jnp.dot(hn, tw[...], preferred_element_type=jnp.float32,
                precision=PREC)
    p = jnp.dot(hn, pw[...], preferred_element_type=jnp.float32,
                precision=PREC)
    t_out[...] = t
    u_out[...] = p - t + bias[...]


def _combine_body(m, u, h, scale, shift, z_out, s_out, q_out):
    hres = h[...] * scale[...] + shift[...]
    mg = m[...]
    agg = jnp.where(jnp.isneginf(mg), 0.0, mg + u[...])
    z = jnp.maximum(agg + hres, 0.0)
    z_out[...] = z

    @pl.when(pl.program_id(0) == 0)
    def _():
        s_out[...] = jnp.zeros_like(s_out)
        q_out[...] = jnp.zeros_like(q_out)

    s_out[...] += jnp.sum(z, axis=0, keepdims=True)
    q_out[...] += jnp.sum(z * z, axis=0, keepdims=True)


def _mlp_body(scale, shift, h, w, b, y_out, s_out, q_out, *, relu):
    hn = h[...] * scale[...] + shift[...]
    y = jnp.dot(hn, w[...], preferred_element_type=jnp.float32,
                precision=PREC) + b[...]
    if relu:
        y = jnp.maximum(y, 0.0)
    y_out[...] = y

    if s_out is not None:
        @pl.when(pl.program_id(0) == 0)
        def _():
            s_out[...] = jnp.zeros_like(s_out)
            q_out[...] = jnp.zeros_like(q_out)

        s_out[...] += jnp.sum(y, axis=0, keepdims=True)
        q_out[...] += jnp.sum(y * y, axis=0, keepdims=True)


def _row_spec(dcol):
    return pl.BlockSpec((RT, dcol), lambda i: (i, 0))


def _full_spec(shape):
    return pl.BlockSpec(shape, lambda i: (0,) * len(shape))


def _pre_call(scale, shift, h, tw, pw, bias):
    return pl.pallas_call(
        _pre_body,
        grid=(N // RT,),
        in_specs=[_full_spec((1, D)), _full_spec((1, D)), _row_spec(D),
                  _full_spec((D, D)), _full_spec((D, D)), _full_spec((1, D))],
        out_specs=[_row_spec(D), _row_spec(D)],
        out_shape=[jax.ShapeDtypeStruct((N, D), jnp.float32)] * 2,
    )(scale, shift, h, tw, pw, bias)


def _combine_call(m, u, h, scale, shift):
    return pl.pallas_call(
        _combine_body,
        grid=(N // RT,),
        in_specs=[_row_spec(D), _row_spec(D), _row_spec(D),
                  _full_spec((1, D)), _full_spec((1, D))],
        out_specs=[_row_spec(D), _full_spec((1, D)), _full_spec((1, D))],
        out_shape=[jax.ShapeDtypeStruct((N, D), jnp.float32),
                   jax.ShapeDtypeStruct((1, D), jnp.float32),
                   jax.ShapeDtypeStruct((1, D), jnp.float32)],
    )(m, u, h, scale, shift)


def _mlp_call(scale, shift, h, w, b, *, relu, stats):
    di, do = w.shape
    if stats:
        body = functools.partial(_mlp_body, relu=relu)
        out_specs = [_row_spec(do), _full_spec((1, do)), _full_spec((1, do))]
        out_shape = [jax.ShapeDtypeStruct((N, do), jnp.float32),
                     jax.ShapeDtypeStruct((1, do), jnp.float32),
                     jax.ShapeDtypeStruct((1, do), jnp.float32)]
    else:
        def body(scale, shift, h, w, b, y_out):
            _mlp_body(scale, shift, h, w, b, y_out, None, None, relu=relu)
        out_specs = [_row_spec(do)]
        out_shape = [jax.ShapeDtypeStruct((N, do), jnp.float32)]
    return pl.pallas_call(
        body,
        grid=(N // RT,),
        in_specs=[_full_spec((1, di)), _full_spec((1, di)), _row_spec(di),
                  _full_spec((di, do)), _full_spec((1, do))],
        out_specs=out_specs,
        out_shape=out_shape,
    )(scale, shift, h, w, b)


def _bn_affine(s, q, g, b):
    mu = s / N
    var = q / N - mu * mu
    scale = g.reshape(1, -1) / jnp.sqrt(var + EPS)
    shift = b.reshape(1, -1) - mu * scale
    return scale, shift


def kernel(x, xyz, params, edge_index):
    srcdst = (edge_index.reshape(2, NCH, CHUNK)
              .transpose(1, 0, 2).reshape(2 * NCH, CHUNK))
    ls, ld, cnts = _partition_kernel()(srcdst)

    ones = jnp.ones((1, D), jnp.float32)
    zeros = jnp.zeros((1, D), jnp.float32)

    h = x
    scale, shift = ones, zeros
    for i in range(2):
        bias = (params[f"theta_b{i}"] + params[f"phi_b{i}"]).reshape(1, D)
        t, u = _pre_call(scale, shift, h, params[f"theta_w{i}"],
                         params[f"phi_w{i}"], bias)
        m = _segmax_kernel()(t, ls, ld, cnts)
        z, s, q = _combine_call(m, u, h, scale, shift)
        scale, shift = _bn_affine(s, q, params[f"bn_g{i}"],
                                  params[f"bn_b{i}"])
        h = z

    a, s, q = _mlp_call(scale, shift, h, params["l1_w"],
                        params["l1_b"].reshape(1, -1), relu=True, stats=True)
    scale, shift = _bn_affine(s, q, params["g1"], params["be1"])
    a, s, q = _mlp_call(scale, shift, a, params["l2_w"],
                        params["l2_b"].reshape(1, -1), relu=True, stats=True)
    scale, shift = _bn_affine(s, q, params["g2"], params["be2"])
    a, s, q = _mlp_call(scale, shift, a, params["l3_w"],
                        params["l3_b"].reshape(1, -1), relu=True, stats=True)
    scale, shift = _bn_affine(s, q, params["g3"], params["be3"])
    (y,) = _mlp_call(scale, shift, a, params["l4_w"],
                     params["l4_b"].reshape(1, -1), relu=False, stats=False)

    out = y.reshape(B, -1, 256).transpose(0, 2, 1)
    return (out, xyz.reshape(B, -1, 3))
